# baseline (speedup 1.0000x reference)
import functools

import jax
import jax.numpy as jnp
from jax import lax
from jax.experimental import pallas as pl
from jax.experimental.pallas import tpu as pltpu
from jax.experimental.pallas import tpu_sc as plsc

G, N, E = 8, 2048, 32768
N_ITER = 5
NC, NS, L = 2, 16, 16
NW = NC * NS
WPG = NW // G
EW = E // WPG
CH = 2048
NCH = E // CH

@functools.cache
def _get_mesh():
    return plsc.VectorSubcoreMesh(core_axis_name="c", subcore_axis_name="s",
                                  num_cores=NC, num_subcores=NS)


GPC = G // NC
EPW = E // NS
RSPAN = 512
NPASS = N // RSPAN
TAB = RSPAN * N
DUMMY = TAB


@functools.cache
def _get_sc_dedup():
    return functools.partial(
        pl.kernel,
        out_type=(
            jax.ShapeDtypeStruct((G, E), jnp.float32),
            jax.ShapeDtypeStruct((G, NS, N), jnp.float32),
            jax.ShapeDtypeStruct((G, NS, N), jnp.float32),
        ),
        mesh=_get_mesh(),
        compiler_params=pltpu.CompilerParams(needs_layout_passes=False),
        scratch_types=[
            pltpu.VMEM_SHARED((TAB + 16,), jnp.int32),
            pltpu.VMEM((EPW,), jnp.int32),
            pltpu.VMEM((EPW,), jnp.int32),
            pltpu.VMEM((EPW,), jnp.int32),
            pltpu.VMEM((EPW,), jnp.int32),
            pltpu.VMEM((EPW,), jnp.int32),
            pltpu.VMEM((EPW,), jnp.float32),
            pltpu.VMEM((N,), jnp.float32),
            pltpu.VMEM((N,), jnp.float32),
            pltpu.VMEM((N,), jnp.float32),
            pltpu.SemaphoreType.DMA,
        ],
    )(_sc_dedup_body)


def _sc_dedup_body(eids_hbm, rows_hbm, cols_hbm, lab_hbm,
                   w_hbm, deg_hbm, s0_hbm,
                   tab_spm, keys_v, eids_v, win_v, rows_v, cols_v, w_v,
                   lab_v, deg_v, s0_v, sem):
    c = lax.axis_index("c")
    s = lax.axis_index("s")
    pltpu.sync_copy(eids_hbm.at[pl.ds(s * EPW, EPW)], eids_v)

    for gg in range(GPC):
        g = c * GPC + gg
        pltpu.sync_copy(rows_hbm.at[g, pl.ds(s * EPW, EPW)], rows_v)
        pltpu.sync_copy(cols_hbm.at[g, pl.ds(s * EPW, EPW)], cols_v)
        pltpu.sync_copy(lab_hbm.at[g], lab_v)

        def zero_body(i, _):
            z = jnp.zeros((L,), jnp.float32)
            deg_v[pl.ds(i * L, L)] = z
            s0_v[pl.ds(i * L, L)] = z
            return 0

        lax.fori_loop(0, N // L, zero_body, 0)

        for p in range(NPASS):
            lo = p * RSPAN

            def key_body(j, _):
                for u in range(8):
                    o = j * 128 + u * L
                    r16 = rows_v[pl.ds(o, L)]
                    c16 = cols_v[pl.ds(o, L)]
                    inhalf = (r16 >= lo) & (r16 < lo + RSPAN)
                    key = (r16 - lo) * N + c16
                    keys_v[pl.ds(o, L)] = jnp.where(inhalf, key, DUMMY)
                return 0

            lax.fori_loop(0, EPW // 128, key_body, 0)

            pltpu.async_copy(eids_v, tab_spm.at[keys_v], sem).wait()
            plsc.subcore_barrier()
            pltpu.async_copy(tab_spm.at[keys_v], win_v, sem).wait()

            def cmp_body(j, _):
                for u in range(8):
                    o = j * 128 + u * L
                    r16 = rows_v[pl.ds(o, L)]
                    c16 = cols_v[pl.ds(o, L)]
                    inhalf = (r16 >= lo) & (r16 < lo + RSPAN)
                    e16 = eids_v[pl.ds(o, L)]
                    v16 = win_v[pl.ds(o, L)]
                    wl = jnp.where((e16 == v16) & inhalf, 1.0, 0.0)
                    prev = w_v[pl.ds(o, L)]
                    w_v[pl.ds(o, L)] = jnp.where(inhalf, wl, prev) \
                        if p else wl
                    plsc.addupdate_scatter(deg_v, [r16], wl)
                    lbl = plsc.load_gather(lab_v, [c16])
                    plsc.addupdate_scatter(s0_v, [r16], lbl * wl)
                return 0

            lax.fori_loop(0, EPW // 128, cmp_body, 0)
            plsc.subcore_barrier()

        pltpu.sync_copy(w_v, w_hbm.at[g, pl.ds(s * EPW, EPW)])
        pltpu.sync_copy(deg_v, deg_hbm.at[g, s])
        pltpu.sync_copy(s0_v, s0_hbm.at[g, s])


@functools.cache
def _get_sc_segsum():
    return functools.partial(
        pl.kernel,
        out_type=jax.ShapeDtypeStruct((G, WPG, N), jnp.float32),
        mesh=_get_mesh(),
        compiler_params=pltpu.CompilerParams(needs_layout_passes=False),
        scratch_types=[
            pltpu.VMEM((N,), jnp.float32),
            pltpu.VMEM((N,), jnp.float32),
            pltpu.VMEM((EW,), jnp.int32),
            pltpu.VMEM((EW,), jnp.int32),
            pltpu.VMEM((EW,), jnp.float32),
        ],
    )(_sc_segsum_body)


def _sc_segsum_body(rows_hbm, cols_hbm, w_hbm, lab_hbm, spart_hbm,
                    lab_v, s_v, rows_v, cols_v, w_v):
    wid = lax.axis_index("c") * NS + lax.axis_index("s")
    g = wid // WPG
    k = wid % WPG
    pltpu.sync_copy(lab_hbm.at[g], lab_v)
    pltpu.sync_copy(rows_hbm.at[g, pl.ds(k * EW, EW)], rows_v)
    pltpu.sync_copy(cols_hbm.at[g, pl.ds(k * EW, EW)], cols_v)
    pltpu.sync_copy(w_hbm.at[g, pl.ds(k * EW, EW)], w_v)

    def zero_body(i, _):
        s_v[pl.ds(i * L, L)] = jnp.zeros((L,), jnp.float32)
        return 0

    lax.fori_loop(0, N // L, zero_body, 0)

    def step(s, _):
        for u in range(8):
            o = s * 128 + u * L
            r16 = rows_v[pl.ds(o, L)]
            c16 = cols_v[pl.ds(o, L)]
            w16 = w_v[pl.ds(o, L)]
            lbl = plsc.load_gather(lab_v, [c16])
            plsc.addupdate_scatter(s_v, [r16], lbl * w16)
        return 0

    lax.fori_loop(0, EW // 128, step, 0)
    pltpu.sync_copy(s_v, spart_hbm.at[g, k])


def _tc_step_body(with_init, l_ref, s_ref, deg_ref, w_ref, f_ref,
                  lnext_ref, fout_ref, eye_ref):
    w00 = w_ref[0, 0]
    w01 = w_ref[0, 1]
    l_row = l_ref[0]
    deg_row = jnp.sum(deg_ref[0], axis=0, keepdims=True)
    s_row = jnp.sum(s_ref[0], axis=0, keepdims=True)
    mb = jnp.max(deg_row)
    a = mb * w00
    h_row = a * l_row + w01 * s_row - w01 * (mb - deg_row)

    ii = lax.broadcasted_iota(jnp.int32, (N, N), 0)
    jj = lax.broadcasted_iota(jnp.int32, (N, N), 1)
    eye_ref[...] = jnp.where(ii == jj, 1.0, 0.0)
    h_col = lax.dot_general(eye_ref[...], h_row, (((1,), (1,)), ((), ())),
                            preferred_element_type=jnp.float32)

    eq_lower = jnp.where((h_col == h_row) & (ii < jj), 1.0, 0.0)
    dup = jnp.sum(eq_lower, axis=0, keepdims=True)
    first = jnp.where(dup == 0.0, 1.0, 0.0)
    lt = jnp.where(h_row < h_col, 1.0, 0.0)
    rank = jnp.sum(lt * first, axis=1, keepdims=True)

    jjf = lax.broadcasted_iota(jnp.int32, (1, N), 1).astype(jnp.float32)
    cnt = jnp.sum(jnp.where(rank == jjf, 1.0, 0.0), axis=0, keepdims=True)
    if with_init:
        l_col = lax.dot_general(eye_ref[...], l_row,
                                (((1,), (1,)), ((), ())),
                                preferred_element_type=jnp.float32)
        cnt0 = jnp.sum(jnp.where(l_col == jjf, 1.0, 0.0), axis=0,
                       keepdims=True)
        fout_ref[...] = (cnt + cnt0).reshape(1, 1, N)
    else:
        fout_ref[...] = (f_ref[0] + cnt).reshape(1, 1, N)
    lnext_ref[...] = rank.reshape(1, 1, N)


def _tc_step(l_flat, spart, degpart, W, f_in, with_init):
    body = functools.partial(_tc_step_body, with_init)
    row3 = pl.BlockSpec((1, 1, N), lambda g: (g, 0, 0))
    ps, pd = spart.shape[1], degpart.shape[1]
    lnext, fout = pl.pallas_call(
        body,
        grid=(G,),
        in_specs=[
            row3,
            pl.BlockSpec((1, ps, N), lambda g: (g, 0, 0)),
            pl.BlockSpec((1, pd, N), lambda g: (g, 0, 0)),
            pl.BlockSpec((1, 2), lambda g: (0, 0)),
            row3,
        ],
        out_specs=[row3, row3],
        out_shape=[
            jax.ShapeDtypeStruct((G, 1, N), jnp.float32),
            jax.ShapeDtypeStruct((G, 1, N), jnp.float32),
        ],
        scratch_shapes=[pltpu.VMEM((N, N), jnp.float32)],
    )(l_flat.reshape(G, 1, N), spart, degpart, W, f_in.reshape(G, 1, N))
    return lnext.reshape(G, N), fout.reshape(G, N)


def _tc_gram_body(f_ref, k_ref):
    F = f_ref[...]
    K0 = lax.dot_general(F, F, (((1,), (1,)), ((), ())),
                         preferred_element_type=jnp.float32)
    ii = lax.broadcasted_iota(jnp.int32, (G, G), 0)
    jj = lax.broadcasted_iota(jnp.int32, (G, G), 1)
    eye = jnp.where(ii == jj, 1.0, 0.0)
    dr = jnp.sqrt(jnp.sum(K0 * eye, axis=0, keepdims=True))
    dc = jnp.sqrt(jnp.sum(K0 * eye, axis=1, keepdims=True))
    k_ref[...] = K0 / (dr * dc)


def kernel(adj_indices, labels, W):
    adj = adj_indices.astype(jnp.int32)
    rows = adj[:, 0, :]
    cols = adj[:, 1, :]
    eids = jnp.arange(E, dtype=jnp.int32)
    lab0 = labels.astype(jnp.float32)

    w, degpart, s0part = _get_sc_dedup()(eids, rows, cols, lab0)

    zero_f = jnp.zeros((G, N), jnp.float32)
    l_cur, f_acc = _tc_step(lab0, s0part, degpart, W, zero_f, with_init=True)
    for _ in range(N_ITER - 1):
        spart = _get_sc_segsum()(rows, cols, w, l_cur)
        l_cur, f_acc = _tc_step(l_cur, spart, degpart, W, f_acc,
                                with_init=False)

    K = pl.pallas_call(
        _tc_gram_body,
        out_shape=jax.ShapeDtypeStruct((G, G), jnp.float32),
    )(f_acc)
    return K


# speedup vs baseline: 1.5161x; 1.5161x over previous
import functools

import jax
import jax.numpy as jnp
from jax import lax
from jax.experimental import pallas as pl
from jax.experimental.pallas import tpu as pltpu
from jax.experimental.pallas import tpu_sc as plsc

G, N, E = 8, 2048, 32768
N_ITER = 5
NC, NS, L = 2, 16, 16
NW = NC * NS
WPG = NW // G
EW = E // WPG
CH = 2048
NCH = E // CH

@functools.cache
def _get_mesh():
    return plsc.VectorSubcoreMesh(core_axis_name="c", subcore_axis_name="s",
                                  num_cores=NC, num_subcores=NS)


NROWW = EW // 128


@functools.cache
def _get_sc_dedup():
    return functools.partial(
        pl.kernel,
        out_type=(
            jax.ShapeDtypeStruct((G, E), jnp.float32),
            jax.ShapeDtypeStruct((G, WPG, N), jnp.float32),
            jax.ShapeDtypeStruct((G, WPG, N), jnp.float32),
            jax.ShapeDtypeStruct((G, WPG, N), jnp.float32),
            jax.ShapeDtypeStruct((G * N * N,), jnp.int32),
        ),
        mesh=_get_mesh(),
        compiler_params=pltpu.CompilerParams(needs_layout_passes=False),
        scratch_types=[
            pltpu.VMEM((EW,), jnp.int32),
            pltpu.VMEM((EW,), jnp.int32),
            pltpu.VMEM((EW,), jnp.int32),
            pltpu.VMEM((EW,), jnp.int32),
            pltpu.VMEM((EW,), jnp.int32),
            pltpu.VMEM((EW,), jnp.float32),
            pltpu.VMEM((N,), jnp.float32),
            pltpu.VMEM((N,), jnp.float32),
            pltpu.VMEM((N,), jnp.float32),
            pltpu.VMEM((N,), jnp.float32),
            pltpu.SemaphoreType.DMA,
        ],
    )(_sc_dedup_body)


def _sc_dedup_body(keys_hbm, eids_hbm, rows_hbm, cols_hbm, lab_hbm,
                   w_hbm, deg_hbm, s0_hbm, bc_hbm, big_hbm,
                   keys_v, eids_v, win_v, rows_v, cols_v, wch_v, lab_v,
                   deg_v, s0_v, bc_v, sem):
    wid = lax.axis_index("c") * NS + lax.axis_index("s")
    g = wid // WPG
    k = wid % WPG
    pltpu.sync_copy(keys_hbm.at[g, pl.ds(k * EW, EW)], keys_v)
    pltpu.sync_copy(eids_hbm.at[pl.ds(k * EW, EW)], eids_v)
    pltpu.sync_copy(lab_hbm.at[g], lab_v)

    pltpu.async_copy(eids_v, big_hbm.at[keys_v], sem).wait()
    plsc.subcore_barrier()
    pltpu.async_copy(big_hbm.at[keys_v], win_v, sem).wait()

    pltpu.sync_copy(rows_hbm.at[g, pl.ds(k * EW, EW)], rows_v)
    pltpu.sync_copy(cols_hbm.at[g, pl.ds(k * EW, EW)], cols_v)

    def zero_body(i, _):
        z = jnp.zeros((L,), jnp.float32)
        deg_v[pl.ds(i * L, L)] = z
        s0_v[pl.ds(i * L, L)] = z
        bc_v[pl.ds(i * L, L)] = z
        return 0

    lax.fori_loop(0, N // L, zero_body, 0)

    NSL = N // WPG
    ones16 = jnp.ones((L,), jnp.float32)

    def bc_body(i, _):
        l16 = lab_v[pl.ds(k * NSL + i * L, L)].astype(jnp.int32)
        plsc.addupdate_scatter(bc_v, [l16], ones16)
        return 0

    lax.fori_loop(0, NSL // L, bc_body, 0)

    def row_body(r, _):
        for u in range(8):
            o = r * 128 + u * L
            e16 = eids_v[pl.ds(o, L)]
            v16 = win_v[pl.ds(o, L)]
            wl = jnp.where(e16 == v16, 1.0, 0.0)
            wch_v[pl.ds(o, L)] = wl
            r16 = rows_v[pl.ds(o, L)]
            c16 = cols_v[pl.ds(o, L)]
            plsc.addupdate_scatter(deg_v, [r16], wl)
            lbl = plsc.load_gather(lab_v, [c16])
            plsc.addupdate_scatter(s0_v, [r16], lbl * wl)
        return 0

    lax.fori_loop(0, EW // 128, row_body, 0)
    pltpu.sync_copy(wch_v, w_hbm.at[g, pl.ds(k * EW, EW)])
    pltpu.sync_copy(deg_v, deg_hbm.at[g, k])
    pltpu.sync_copy(s0_v, s0_hbm.at[g, k])
    pltpu.sync_copy(bc_v, bc_hbm.at[g, k])


@functools.cache
def _get_sc_segsum():
    return functools.partial(
        pl.kernel,
        out_type=(
            jax.ShapeDtypeStruct((G, WPG, N), jnp.float32),
            jax.ShapeDtypeStruct((G, WPG, N), jnp.float32),
        ),
        mesh=_get_mesh(),
        compiler_params=pltpu.CompilerParams(needs_layout_passes=False),
        scratch_types=[
            pltpu.VMEM((N,), jnp.float32),
            pltpu.VMEM((N,), jnp.float32),
            pltpu.VMEM((N,), jnp.float32),
            pltpu.VMEM((EW,), jnp.int32),
            pltpu.VMEM((EW,), jnp.int32),
            pltpu.VMEM((EW,), jnp.float32),
        ],
    )(_sc_segsum_body)


def _sc_segsum_body(rows_hbm, cols_hbm, w_hbm, lab_hbm, spart_hbm, bc_hbm,
                    lab_v, s_v, bc_v, rows_v, cols_v, w_v):
    wid = lax.axis_index("c") * NS + lax.axis_index("s")
    g = wid // WPG
    k = wid % WPG
    pltpu.sync_copy(lab_hbm.at[g], lab_v)
    pltpu.sync_copy(rows_hbm.at[g, pl.ds(k * EW, EW)], rows_v)
    pltpu.sync_copy(cols_hbm.at[g, pl.ds(k * EW, EW)], cols_v)
    pltpu.sync_copy(w_hbm.at[g, pl.ds(k * EW, EW)], w_v)

    def zero_body(i, _):
        z = jnp.zeros((L,), jnp.float32)
        s_v[pl.ds(i * L, L)] = z
        bc_v[pl.ds(i * L, L)] = z
        return 0

    lax.fori_loop(0, N // L, zero_body, 0)

    NSL = N // WPG
    ones16 = jnp.ones((L,), jnp.float32)

    def bc_body(i, _):
        l16 = lab_v[pl.ds(k * NSL + i * L, L)].astype(jnp.int32)
        plsc.addupdate_scatter(bc_v, [l16], ones16)
        return 0

    lax.fori_loop(0, NSL // L, bc_body, 0)

    def step(s, _):
        for u in range(8):
            o = s * 128 + u * L
            r16 = rows_v[pl.ds(o, L)]
            c16 = cols_v[pl.ds(o, L)]
            w16 = w_v[pl.ds(o, L)]
            lbl = plsc.load_gather(lab_v, [c16])
            plsc.addupdate_scatter(s_v, [r16], lbl * w16)
        return 0

    lax.fori_loop(0, EW // 128, step, 0)
    pltpu.sync_copy(s_v, spart_hbm.at[g, k])
    pltpu.sync_copy(bc_v, bc_hbm.at[g, k])


def _tc_step_body(last, l_ref, s_ref, deg_ref, w_ref,
                  lnext_ref, *rest):
    if last:
        fout_ref, eye_ref = rest
    else:
        (eye_ref,) = rest
    w00 = w_ref[0, 0]
    w01 = w_ref[0, 1]
    l_row = l_ref[0]
    deg_row = jnp.sum(deg_ref[0], axis=0, keepdims=True)
    s_row = jnp.sum(s_ref[0], axis=0, keepdims=True)
    mb = jnp.max(deg_row)
    a = mb * w00
    h_row = a * l_row + w01 * s_row - w01 * (mb - deg_row)

    ii = lax.broadcasted_iota(jnp.int32, (N, N), 0)
    jj = lax.broadcasted_iota(jnp.int32, (N, N), 1)
    eye_ref[...] = jnp.where(ii == jj, 1.0, 0.0)
    h_col = lax.dot_general(eye_ref[...], h_row, (((1,), (1,)), ((), ())),
                            preferred_element_type=jnp.float32)

    eqcnt = jnp.sum(jnp.where(h_col == h_row, 1.0, 0.0), axis=0,
                    keepdims=True)
    invc = 1.0 / eqcnt
    acc = jnp.sum(jnp.where(h_row < h_col, invc, 0.0), axis=1,
                  keepdims=True)
    rank = jnp.floor(acc + 0.5)
    if last:
        jjf = lax.broadcasted_iota(jnp.int32, (1, N), 1).astype(jnp.float32)
        cnt = jnp.sum(jnp.where(rank == jjf, 1.0, 0.0), axis=0,
                      keepdims=True)
        fout_ref[...] = cnt.reshape(1, 1, N)
    lnext_ref[...] = rank.reshape(1, 1, N)


def _tc_step(l_flat, spart, degpart, W, last):
    body = functools.partial(_tc_step_body, last)
    row3 = pl.BlockSpec((1, 1, N), lambda g: (g, 0, 0))
    ps, pd = spart.shape[1], degpart.shape[1]
    out_specs = [row3, row3] if last else [row3]
    out_shape = [jax.ShapeDtypeStruct((G, 1, N), jnp.float32)] * (
        2 if last else 1)
    outs = pl.pallas_call(
        body,
        grid=(G,),
        in_specs=[
            row3,
            pl.BlockSpec((1, ps, N), lambda g: (g, 0, 0)),
            pl.BlockSpec((1, pd, N), lambda g: (g, 0, 0)),
            pl.BlockSpec((1, 2), lambda g: (0, 0)),
        ],
        out_specs=out_specs,
        out_shape=out_shape,
        scratch_shapes=[pltpu.VMEM((N, N), jnp.float32)],
    )(l_flat.reshape(G, 1, N), spart, degpart, W)
    if last:
        return outs[0].reshape(G, N), outs[1].reshape(G, N)
    return outs[0].reshape(G, N), None


def _tc_gram_body(bc_ref, k_ref):
    F = jnp.sum(bc_ref[...], axis=1)
    K0 = lax.dot_general(F, F, (((1,), (1,)), ((), ())),
                         preferred_element_type=jnp.float32)
    ii = lax.broadcasted_iota(jnp.int32, (G, G), 0)
    jj = lax.broadcasted_iota(jnp.int32, (G, G), 1)
    eye = jnp.where(ii == jj, 1.0, 0.0)
    dr = jnp.sqrt(jnp.sum(K0 * eye, axis=0, keepdims=True))
    dc = jnp.sqrt(jnp.sum(K0 * eye, axis=1, keepdims=True))
    k_ref[...] = K0 / (dr * dc)


def kernel(adj_indices, labels, W):
    adj = adj_indices.astype(jnp.int32)
    rows = adj[:, 0, :]
    cols = adj[:, 1, :]
    gofs = (jnp.arange(G, dtype=jnp.int32) * (N * N))[:, None]
    keys = gofs + rows * N + cols
    eids = jnp.arange(E, dtype=jnp.int32)
    lab0 = labels.astype(jnp.float32)

    w, degpart, s0part, bc0, _ = _get_sc_dedup()(keys, eids, rows, cols,
                                                 lab0)

    bcs = [bc0]
    l_cur, _ = _tc_step(lab0, s0part, degpart, W, last=False)
    for it in range(N_ITER - 1):
        spart, bct = _get_sc_segsum()(rows, cols, w, l_cur)
        bcs.append(bct)
        l_cur, cnt_last = _tc_step(l_cur, spart, degpart, W,
                                   last=(it == N_ITER - 2))
    bcs.append(cnt_last.reshape(G, 1, N))

    K = pl.pallas_call(
        _tc_gram_body,
        out_shape=jax.ShapeDtypeStruct((G, G), jnp.float32),
    )(jnp.concatenate(bcs, axis=1))
    return K


# speedup vs baseline: 1.5545x; 1.0253x over previous
import functools

import jax
import jax.numpy as jnp
from jax import lax
from jax.experimental import pallas as pl
from jax.experimental.pallas import tpu as pltpu
from jax.experimental.pallas import tpu_sc as plsc

G, N, E = 8, 2048, 32768
N_ITER = 5
NC, NS, L = 2, 16, 16
NW = NC * NS
WPG = NW // G
EW = E // WPG
CH = 2048
NCH = E // CH

@functools.cache
def _get_mesh():
    return plsc.VectorSubcoreMesh(core_axis_name="c", subcore_axis_name="s",
                                  num_cores=NC, num_subcores=NS)


NROWW = EW // 128


@functools.cache
def _get_sc_dedup():
    return functools.partial(
        pl.kernel,
        out_type=(
            jax.ShapeDtypeStruct((G, E), jnp.float32),
            jax.ShapeDtypeStruct((G, WPG, N), jnp.float32),
            jax.ShapeDtypeStruct((G, WPG, N), jnp.float32),
            jax.ShapeDtypeStruct((G, WPG, N), jnp.float32),
            jax.ShapeDtypeStruct((G * N * N,), jnp.int32),
        ),
        mesh=_get_mesh(),
        compiler_params=pltpu.CompilerParams(needs_layout_passes=False),
        scratch_types=[
            pltpu.VMEM((EW,), jnp.int32),
            pltpu.VMEM((EW,), jnp.int32),
            pltpu.VMEM((EW,), jnp.int32),
            pltpu.VMEM((EW,), jnp.int32),
            pltpu.VMEM((EW,), jnp.int32),
            pltpu.VMEM((EW,), jnp.float32),
            pltpu.VMEM((N,), jnp.float32),
            pltpu.VMEM((N,), jnp.float32),
            pltpu.VMEM((N,), jnp.float32),
            pltpu.VMEM((N,), jnp.float32),
            pltpu.SemaphoreType.DMA,
        ],
    )(_sc_dedup_body)


def _sc_dedup_body(keys_hbm, eids_hbm, rows_hbm, cols_hbm, lab_hbm,
                   w_hbm, deg_hbm, s0_hbm, bc_hbm, big_hbm,
                   keys_v, eids_v, win_v, rows_v, cols_v, wch_v, lab_v,
                   deg_v, s0_v, bc_v, sem):
    wid = lax.axis_index("c") * NS + lax.axis_index("s")
    g = wid // WPG
    k = wid % WPG
    pltpu.sync_copy(keys_hbm.at[g, pl.ds(k * EW, EW)], keys_v)
    pltpu.sync_copy(eids_hbm.at[pl.ds(k * EW, EW)], eids_v)
    pltpu.sync_copy(lab_hbm.at[g], lab_v)

    pltpu.async_copy(eids_v, big_hbm.at[keys_v], sem).wait()
    plsc.subcore_barrier()
    pltpu.async_copy(big_hbm.at[keys_v], win_v, sem).wait()

    pltpu.sync_copy(rows_hbm.at[g, pl.ds(k * EW, EW)], rows_v)
    pltpu.sync_copy(cols_hbm.at[g, pl.ds(k * EW, EW)], cols_v)

    def zero_body(i, _):
        z = jnp.zeros((L,), jnp.float32)
        deg_v[pl.ds(i * L, L)] = z
        s0_v[pl.ds(i * L, L)] = z
        bc_v[pl.ds(i * L, L)] = z
        return 0

    lax.fori_loop(0, N // L, zero_body, 0)

    NSL = N // WPG
    ones16 = jnp.ones((L,), jnp.float32)

    def bc_body(i, _):
        l16 = lab_v[pl.ds(k * NSL + i * L, L)].astype(jnp.int32)
        plsc.addupdate_scatter(bc_v, [l16], ones16)
        return 0

    lax.fori_loop(0, NSL // L, bc_body, 0)

    def row_body(r, _):
        for u in range(8):
            o = r * 128 + u * L
            e16 = eids_v[pl.ds(o, L)]
            v16 = win_v[pl.ds(o, L)]
            wl = jnp.where(e16 == v16, 1.0, 0.0)
            wch_v[pl.ds(o, L)] = wl
            r16 = rows_v[pl.ds(o, L)]
            c16 = cols_v[pl.ds(o, L)]
            plsc.addupdate_scatter(deg_v, [r16], wl)
            lbl = plsc.load_gather(lab_v, [c16])
            plsc.addupdate_scatter(s0_v, [r16], lbl * wl)
        return 0

    lax.fori_loop(0, EW // 128, row_body, 0)
    pltpu.sync_copy(wch_v, w_hbm.at[g, pl.ds(k * EW, EW)])
    pltpu.sync_copy(deg_v, deg_hbm.at[g, k])
    pltpu.sync_copy(s0_v, s0_hbm.at[g, k])
    pltpu.sync_copy(bc_v, bc_hbm.at[g, k])


@functools.cache
def _get_sc_segsum():
    return functools.partial(
        pl.kernel,
        out_type=(
            jax.ShapeDtypeStruct((G, WPG, N), jnp.float32),
            jax.ShapeDtypeStruct((G, WPG, N), jnp.float32),
        ),
        mesh=_get_mesh(),
        compiler_params=pltpu.CompilerParams(needs_layout_passes=False),
        scratch_types=[
            pltpu.VMEM((N,), jnp.float32),
            pltpu.VMEM((N,), jnp.float32),
            pltpu.VMEM((N,), jnp.float32),
            pltpu.VMEM((EW,), jnp.int32),
            pltpu.VMEM((EW,), jnp.int32),
            pltpu.VMEM((EW,), jnp.float32),
        ],
    )(_sc_segsum_body)


def _sc_segsum_body(rows_hbm, cols_hbm, w_hbm, lab_hbm, spart_hbm, bc_hbm,
                    lab_v, s_v, bc_v, rows_v, cols_v, w_v):
    wid = lax.axis_index("c") * NS + lax.axis_index("s")
    g = wid // WPG
    k = wid % WPG
    pltpu.sync_copy(lab_hbm.at[g], lab_v)
    pltpu.sync_copy(rows_hbm.at[g, pl.ds(k * EW, EW)], rows_v)
    pltpu.sync_copy(cols_hbm.at[g, pl.ds(k * EW, EW)], cols_v)
    pltpu.sync_copy(w_hbm.at[g, pl.ds(k * EW, EW)], w_v)

    def zero_body(i, _):
        z = jnp.zeros((L,), jnp.float32)
        s_v[pl.ds(i * L, L)] = z
        bc_v[pl.ds(i * L, L)] = z
        return 0

    lax.fori_loop(0, N // L, zero_body, 0)

    NSL = N // WPG
    ones16 = jnp.ones((L,), jnp.float32)

    def bc_body(i, _):
        l16 = lab_v[pl.ds(k * NSL + i * L, L)].astype(jnp.int32)
        plsc.addupdate_scatter(bc_v, [l16], ones16)
        return 0

    lax.fori_loop(0, NSL // L, bc_body, 0)

    def step(s, _):
        for u in range(8):
            o = s * 128 + u * L
            r16 = rows_v[pl.ds(o, L)]
            c16 = cols_v[pl.ds(o, L)]
            w16 = w_v[pl.ds(o, L)]
            lbl = plsc.load_gather(lab_v, [c16])
            plsc.addupdate_scatter(s_v, [r16], lbl * w16)
        return 0

    lax.fori_loop(0, EW // 128, step, 0)
    pltpu.sync_copy(s_v, spart_hbm.at[g, k])
    pltpu.sync_copy(bc_v, bc_hbm.at[g, k])


def _tc_step_body(last, l_ref, s_ref, deg_ref, w_ref,
                  lnext_ref, *rest):
    if last:
        (fout_ref,) = rest
    w00 = w_ref[0, 0]
    w01 = w_ref[0, 1]
    l_row = l_ref[0]
    deg_row = jnp.sum(deg_ref[0], axis=0, keepdims=True)
    s_row = jnp.sum(s_ref[0], axis=0, keepdims=True)
    mb = jnp.max(deg_row)
    a = mb * w00
    h_row = a * l_row + w01 * s_row - w01 * (mb - deg_row)

    h_col = h_row.reshape(N, 1)

    eqcnt = jnp.sum(jnp.where(h_col == h_row, 1.0, 0.0), axis=0,
                    keepdims=True)
    invc = 1.0 / eqcnt
    acc = jnp.sum(jnp.where(h_row < h_col, invc, 0.0), axis=1,
                  keepdims=True)
    rank = jnp.floor(acc + 0.5)
    if last:
        jjf = lax.broadcasted_iota(jnp.int32, (1, N), 1).astype(jnp.float32)
        cnt = jnp.sum(jnp.where(rank == jjf, 1.0, 0.0), axis=0,
                      keepdims=True)
        fout_ref[...] = cnt.reshape(1, 1, N)
    lnext_ref[...] = rank.reshape(1, 1, N)


def _tc_step(l_flat, spart, degpart, W, last):
    body = functools.partial(_tc_step_body, last)
    row3 = pl.BlockSpec((1, 1, N), lambda g: (g, 0, 0))
    ps, pd = spart.shape[1], degpart.shape[1]
    out_specs = [row3, row3] if last else [row3]
    out_shape = [jax.ShapeDtypeStruct((G, 1, N), jnp.float32)] * (
        2 if last else 1)
    outs = pl.pallas_call(
        body,
        grid=(G,),
        in_specs=[
            row3,
            pl.BlockSpec((1, ps, N), lambda g: (g, 0, 0)),
            pl.BlockSpec((1, pd, N), lambda g: (g, 0, 0)),
            pl.BlockSpec((1, 2), lambda g: (0, 0)),
        ],
        out_specs=out_specs,
        out_shape=out_shape,
    )(l_flat.reshape(G, 1, N), spart, degpart, W)
    if last:
        return outs[0].reshape(G, N), outs[1].reshape(G, N)
    return outs[0].reshape(G, N), None


def _tc_gram_body(bc_ref, k_ref):
    F = jnp.sum(bc_ref[...], axis=1)
    K0 = lax.dot_general(F, F, (((1,), (1,)), ((), ())),
                         preferred_element_type=jnp.float32)
    ii = lax.broadcasted_iota(jnp.int32, (G, G), 0)
    jj = lax.broadcasted_iota(jnp.int32, (G, G), 1)
    eye = jnp.where(ii == jj, 1.0, 0.0)
    dr = jnp.sqrt(jnp.sum(K0 * eye, axis=0, keepdims=True))
    dc = jnp.sqrt(jnp.sum(K0 * eye, axis=1, keepdims=True))
    k_ref[...] = K0 / (dr * dc)


def kernel(adj_indices, labels, W):
    adj = adj_indices.astype(jnp.int32)
    rows = adj[:, 0, :]
    cols = adj[:, 1, :]
    gofs = (jnp.arange(G, dtype=jnp.int32) * (N * N))[:, None]
    keys = gofs + rows * N + cols
    eids = jnp.arange(E, dtype=jnp.int32)
    lab0 = labels.astype(jnp.float32)

    w, degpart, s0part, bc0, _ = _get_sc_dedup()(keys, eids, rows, cols,
                                                 lab0)

    bcs = [bc0]
    l_cur, _ = _tc_step(lab0, s0part, degpart, W, last=False)
    for it in range(N_ITER - 1):
        spart, bct = _get_sc_segsum()(rows, cols, w, l_cur)
        bcs.append(bct)
        l_cur, cnt_last = _tc_step(l_cur, spart, degpart, W,
                                   last=(it == N_ITER - 2))
    bcs.append(cnt_last.reshape(G, 1, N))

    K = pl.pallas_call(
        _tc_gram_body,
        out_shape=jax.ShapeDtypeStruct((G, G), jnp.float32),
    )(jnp.concatenate(bcs, axis=1))
    return K
